# baseline (speedup 1.0000x reference)
import functools

import jax
import jax.numpy as jnp
from jax import lax
from jax.experimental import pallas as pl
from jax.experimental.pallas import tpu as pltpu
from jax.experimental.pallas import tpu_sc as plsc

N = 10000
NP = 10240
T = 64
H = 128
E = 320000
CH = 128
NCH_W = 79
EPW = NCH_W * CH
EP = 32 * EPW
NCH_T = EP // 16 // CH
STRIPE = NP // 16
SFLAT = NP * T
SSTRIPE = SFLAT // 16

_mesh = plsc.VectorSubcoreMesh(core_axis_name="c", subcore_axis_name="s")
_sc_params = pltpu.CompilerParams(needs_layout_passes=False)


@functools.partial(
    pl.kernel,
    out_type=jax.ShapeDtypeStruct((2, NP), jnp.float32),
    mesh=_mesh,
    compiler_params=_sc_params,
    scratch_types=[
        pltpu.VMEM_SHARED((NP,), jnp.float32),
        pltpu.VMEM((NCH_T, CH), jnp.int32),
        pltpu.VMEM((CH,), jnp.float32),
        pltpu.VMEM((STRIPE,), jnp.float32),
    ],
)
def _deg_kernel(edges_hbm, out_hbm, acc, eidx, ones_v, zeros_v):
    cid = lax.axis_index("c")
    sid = lax.axis_index("s")
    for j in range(CH // 16):
        ones_v[pl.ds(16 * j, 16)] = jnp.full((16,), 1.0, jnp.float32)
    for j in range(STRIPE // 16):
        zeros_v[pl.ds(16 * j, 16)] = jnp.zeros((16,), jnp.float32)
    pltpu.sync_copy(zeros_v, acc.at[pl.ds(sid * STRIPE, STRIPE)])
    pltpu.sync_copy(edges_hbm.at[cid, sid], eidx)
    plsc.subcore_barrier()

    def body(c, carry):
        pltpu.sync_copy(ones_v, acc.at[eidx.at[c]], add=True)
        return carry

    lax.fori_loop(0, NCH_T, body, 0)
    plsc.subcore_barrier()
    pltpu.sync_copy(acc.at[pl.ds(sid * STRIPE, STRIPE)],
                    out_hbm.at[cid, pl.ds(sid * STRIPE, STRIPE)])


@functools.partial(
    pl.kernel,
    out_type=jax.ShapeDtypeStruct((2, SFLAT), jnp.float32),
    mesh=_mesh,
    compiler_params=_sc_params,
    scratch_types=[
        pltpu.VMEM_SHARED((SFLAT,), jnp.float32),
        pltpu.VMEM((NP,), jnp.int32),
        pltpu.VMEM((NP,), jnp.float32),
        pltpu.VMEM((EPW,), jnp.int32),
        pltpu.VMEM((EPW,), jnp.int32),
        pltpu.VMEM((NCH_W, CH), jnp.int32),
        pltpu.VMEM((EPW,), jnp.float32),
    ],
)
def _hist_kernel(src_hbm, dst_hbm, nf_hbm, onorm_hbm, z_hbm, out_hbm,
                 s_acc, t_v, w_v, src_v, dst_v, idx_v, val_v):
    cid = lax.axis_index("c")
    sid = lax.axis_index("s")
    wid = cid * 16 + sid
    pltpu.sync_copy(z_hbm, s_acc.at[pl.ds(sid * SSTRIPE, SSTRIPE)])
    pltpu.sync_copy(nf_hbm, t_v)
    pltpu.sync_copy(onorm_hbm, w_v)
    pltpu.sync_copy(src_hbm.at[wid], src_v)
    pltpu.sync_copy(dst_hbm.at[wid], dst_v)
    plsc.subcore_barrier()

    def body(c, carry):
        for j in range(CH // 16):
            s16 = src_v[pl.ds(c * CH + j * 16, 16)]
            d16 = dst_v[pl.ds(c * CH + j * 16, 16)]
            t16 = plsc.load_gather(t_v, [s16])
            w16 = plsc.load_gather(w_v, [s16])
            idx_v[c, pl.ds(j * 16, 16)] = d16 * T + t16
            val_v[pl.ds(c * CH + j * 16, 16)] = w16
        pltpu.sync_copy(val_v.at[pl.ds(c * CH, CH)],
                        s_acc.at[idx_v.at[c]], add=True)
        return carry

    lax.fori_loop(0, NCH_W, body, 0)
    plsc.subcore_barrier()
    pltpu.sync_copy(s_acc.at[pl.ds(sid * SSTRIPE, SSTRIPE)],
                    out_hbm.at[cid, pl.ds(sid * SSTRIPE, SSTRIPE)])


@functools.partial(
    pl.kernel,
    out_type=jax.ShapeDtypeStruct((2, NP, H), jnp.float32),
    mesh=_mesh,
    compiler_params=_sc_params,
    scratch_types=[
        pltpu.VMEM_SHARED((NP, H), jnp.float32),
        pltpu.VMEM((NCH_W, CH), jnp.int32),
        pltpu.VMEM((NCH_W, CH), jnp.int32),
        pltpu.VMEM((CH, H), jnp.float32),
        pltpu.SemaphoreType.DMA,
    ],
)
def _agg_kernel(src_hbm, dst_hbm, h1s_hbm, z_hbm, out_hbm,
                agc, src_v, dst_v, rows_v, sem):
    cid = lax.axis_index("c")
    sid = lax.axis_index("s")
    wid = cid * 16 + sid
    pltpu.sync_copy(z_hbm, agc.at[pl.ds(sid * STRIPE, STRIPE)])
    pltpu.sync_copy(src_hbm.at[wid], src_v)
    pltpu.sync_copy(dst_hbm.at[wid], dst_v)
    plsc.subcore_barrier()

    def body(c, carry):
        pltpu.async_copy(h1s_hbm.at[src_v.at[c]], rows_v, sem).wait()
        pltpu.sync_copy(rows_v, agc.at[dst_v.at[c]], add=True)
        return carry

    lax.fori_loop(0, NCH_W, body, 0)
    plsc.subcore_barrier()
    pltpu.sync_copy(agc.at[pl.ds(sid * STRIPE, STRIPE)],
                    out_hbm.at[cid, pl.ds(sid * STRIPE, STRIPE)])


def _norm_body(deg_ref, out_ref):
    out_ref[...] = lax.rsqrt(jnp.maximum(deg_ref[...], 1.0))


_norm = pl.pallas_call(
    _norm_body,
    out_shape=jax.ShapeDtypeStruct((2, NP), jnp.float32),
)


def _tc1_body(s_ref, e_ref, w1_ref, b1_ref, inorm_ref, onorm_ref, h1s_ref):
    s = s_ref[0] + s_ref[1]
    agg = jnp.dot(s, e_ref[...], preferred_element_type=jnp.float32)
    h = agg * inorm_ref[...]
    h = jnp.dot(h, w1_ref[...], preferred_element_type=jnp.float32) + b1_ref[...]
    h = jnp.maximum(h, 0.0)
    h1s_ref[...] = h * onorm_ref[...]


_tc1 = pl.pallas_call(
    _tc1_body,
    out_shape=jax.ShapeDtypeStruct((NP, H), jnp.float32),
)


def _tc2_body(a_ref, w2_ref, b2_ref, inorm_ref, out_ref):
    a = a_ref[0] + a_ref[1]
    h = a * inorm_ref[...]
    h = jnp.dot(h, w2_ref[...], preferred_element_type=jnp.float32) + b2_ref[...]
    h = jnp.maximum(h, 0.0)
    rows = lax.broadcasted_iota(jnp.int32, (NP, 1), 0)
    mask = (rows < N).astype(jnp.float32)
    out_ref[...] = jnp.sum(h * mask, axis=0, keepdims=True) * (1.0 / N)


_tc2 = pl.pallas_call(
    _tc2_body,
    out_shape=jax.ShapeDtypeStruct((1, H), jnp.float32),
)


def kernel(node_feat, edge_index, emb_table, W1, b1, W2, b2):
    src = edge_index[0].astype(jnp.int32)
    dst = edge_index[1].astype(jnp.int32)
    src_p = jnp.pad(src, (0, EP - E), constant_values=N)
    dst_pad = N + (jnp.arange(EP - E, dtype=jnp.int32) % (NP - N))
    dst_p = jnp.concatenate([dst, dst_pad])
    nf_p = jnp.pad(node_feat.astype(jnp.int32), (0, NP - N))

    edges_a = jnp.stack([src_p, dst_p]).reshape(2, 16, NCH_T, CH)
    deg = _deg_kernel(edges_a)
    norms = _norm(deg)
    onorm = norms[0]
    inorm = norms[1].reshape(NP, 1)

    src_w = src_p.reshape(32, EPW)
    dst_w = dst_p.reshape(32, EPW)
    zeros_s = jnp.zeros((SSTRIPE,), jnp.float32)
    s_parts = _hist_kernel(src_w, dst_w, nf_p, onorm, zeros_s)
    s_parts = s_parts.reshape(2, NP, T)

    h1s = _tc1(s_parts, emb_table, W1, b1.reshape(1, H), inorm,
               onorm.reshape(NP, 1))

    src_c = src_p.reshape(32, NCH_W, CH)
    dst_c = dst_p.reshape(32, NCH_W, CH)
    zeros_nd = jnp.zeros((STRIPE, H), jnp.float32)
    agg = _agg_kernel(src_c, dst_c, h1s, zeros_nd)

    return _tc2(agg, W2, b2.reshape(1, H), inorm)


# speedup vs baseline: 1.6794x; 1.6794x over previous
import functools

import jax
import jax.numpy as jnp
from jax import lax
from jax.experimental import pallas as pl
from jax.experimental.pallas import tpu as pltpu
from jax.experimental.pallas import tpu_sc as plsc

N = 10000
NP = 10240
T = 64
H = 128
E = 320000
CH = 128
NCH_W = 79
EPW = NCH_W * CH
EP = 32 * EPW
NCH_T = EP // 16 // CH
STRIPE = NP // 16
SFLAT = NP * T
SSTRIPE = SFLAT // 16

_mesh = plsc.VectorSubcoreMesh(core_axis_name="c", subcore_axis_name="s")
_sc_params = pltpu.CompilerParams(needs_layout_passes=False)


@functools.partial(
    pl.kernel,
    out_type=jax.ShapeDtypeStruct((2, NP), jnp.float32),
    mesh=_mesh,
    compiler_params=_sc_params,
    scratch_types=[
        pltpu.VMEM_SHARED((NP,), jnp.float32),
        pltpu.VMEM((NCH_T, CH), jnp.int32),
        pltpu.VMEM((CH,), jnp.float32),
        pltpu.VMEM((STRIPE,), jnp.float32),
    ],
)
def _deg_kernel(edges_hbm, out_hbm, acc, eidx, ones_v, zeros_v):
    cid = lax.axis_index("c")
    sid = lax.axis_index("s")
    for j in range(CH // 16):
        ones_v[pl.ds(16 * j, 16)] = jnp.full((16,), 1.0, jnp.float32)
    for j in range(STRIPE // 16):
        zeros_v[pl.ds(16 * j, 16)] = jnp.zeros((16,), jnp.float32)
    pltpu.sync_copy(zeros_v, acc.at[pl.ds(sid * STRIPE, STRIPE)])
    pltpu.sync_copy(edges_hbm.at[cid, sid], eidx)
    plsc.subcore_barrier()

    def body(c, carry):
        pltpu.sync_copy(ones_v, acc.at[eidx.at[c]], add=True)
        return carry

    lax.fori_loop(0, NCH_T, body, 0)
    plsc.subcore_barrier()
    pltpu.sync_copy(acc.at[pl.ds(sid * STRIPE, STRIPE)],
                    out_hbm.at[cid, pl.ds(sid * STRIPE, STRIPE)])


@functools.partial(
    pl.kernel,
    out_type=jax.ShapeDtypeStruct((2, SFLAT), jnp.float32),
    mesh=_mesh,
    compiler_params=_sc_params,
    scratch_types=[
        pltpu.VMEM_SHARED((SFLAT,), jnp.float32),
        pltpu.VMEM((NP,), jnp.int32),
        pltpu.VMEM((NP,), jnp.float32),
        pltpu.VMEM((EPW,), jnp.int32),
        pltpu.VMEM((EPW,), jnp.int32),
        pltpu.VMEM((NCH_W, CH), jnp.int32),
        pltpu.VMEM((EPW,), jnp.float32),
    ],
)
def _hist_kernel(src_hbm, dst_hbm, nf_hbm, onorm_hbm, z_hbm, out_hbm,
                 s_acc, t_v, w_v, src_v, dst_v, idx_v, val_v):
    cid = lax.axis_index("c")
    sid = lax.axis_index("s")
    wid = cid * 16 + sid
    pltpu.sync_copy(z_hbm, s_acc.at[pl.ds(sid * SSTRIPE, SSTRIPE)])
    pltpu.sync_copy(nf_hbm, t_v)
    pltpu.sync_copy(onorm_hbm, w_v)
    pltpu.sync_copy(src_hbm.at[wid], src_v)
    pltpu.sync_copy(dst_hbm.at[wid], dst_v)
    plsc.subcore_barrier()

    def body(c, carry):
        for j in range(CH // 16):
            s16 = src_v[pl.ds(c * CH + j * 16, 16)]
            d16 = dst_v[pl.ds(c * CH + j * 16, 16)]
            t16 = plsc.load_gather(t_v, [s16])
            w16 = plsc.load_gather(w_v, [s16])
            idx_v[c, pl.ds(j * 16, 16)] = d16 * T + t16
            val_v[pl.ds(c * CH + j * 16, 16)] = w16
        pltpu.sync_copy(val_v.at[pl.ds(c * CH, CH)],
                        s_acc.at[idx_v.at[c]], add=True)
        return carry

    lax.fori_loop(0, NCH_W, body, 0)
    plsc.subcore_barrier()
    pltpu.sync_copy(s_acc.at[pl.ds(sid * SSTRIPE, SSTRIPE)],
                    out_hbm.at[cid, pl.ds(sid * SSTRIPE, SSTRIPE)])


@functools.partial(
    pl.kernel,
    out_type=jax.ShapeDtypeStruct((2, NP, H), jnp.float32),
    mesh=_mesh,
    compiler_params=_sc_params,
    scratch_types=[
        pltpu.VMEM_SHARED((NP, H), jnp.float32),
        pltpu.VMEM((NCH_W, CH), jnp.int32),
        pltpu.VMEM((NCH_W, CH), jnp.int32),
        pltpu.VMEM((CH, H), jnp.float32),
        pltpu.SemaphoreType.DMA,
    ],
)
def _agg_kernel(src_hbm, dst_hbm, h1s_hbm, z_hbm, out_hbm,
                agc, src_v, dst_v, rows_v, sem):
    cid = lax.axis_index("c")
    sid = lax.axis_index("s")
    wid = cid * 16 + sid
    pltpu.sync_copy(z_hbm, agc.at[pl.ds(sid * STRIPE, STRIPE)])
    pltpu.sync_copy(src_hbm.at[wid], src_v)
    pltpu.sync_copy(dst_hbm.at[wid], dst_v)
    plsc.subcore_barrier()

    def body(c, carry):
        pltpu.async_copy(h1s_hbm.at[src_v.at[c]], rows_v, sem).wait()
        pltpu.sync_copy(rows_v, agc.at[dst_v.at[c]], add=True)
        return carry

    lax.fori_loop(0, NCH_W, body, 0)
    plsc.subcore_barrier()
    pltpu.sync_copy(agc.at[pl.ds(sid * STRIPE, STRIPE)],
                    out_hbm.at[cid, pl.ds(sid * STRIPE, STRIPE)])


def _norm_body(deg_ref, out_ref):
    out_ref[...] = lax.rsqrt(jnp.maximum(deg_ref[...], 1.0))


_norm = pl.pallas_call(
    _norm_body,
    out_shape=jax.ShapeDtypeStruct((2, NP), jnp.float32),
)


def _tc1_body(s_ref, e_ref, w1_ref, b1_ref, inorm_ref, onorm_ref, h1s_ref):
    s = s_ref[0] + s_ref[1]
    agg = jnp.dot(s, e_ref[...], preferred_element_type=jnp.float32)
    h = agg * inorm_ref[...]
    h = jnp.dot(h, w1_ref[...], preferred_element_type=jnp.float32) + b1_ref[...]
    h = jnp.maximum(h, 0.0)
    h1s_ref[...] = h * onorm_ref[...]


_tc1 = pl.pallas_call(
    _tc1_body,
    out_shape=jax.ShapeDtypeStruct((NP, H), jnp.float32),
)


def _tc2_body(a_ref, w2_ref, b2_ref, inorm_ref, out_ref):
    a = a_ref[0] + a_ref[1]
    h = a * inorm_ref[...]
    h = jnp.dot(h, w2_ref[...], preferred_element_type=jnp.float32) + b2_ref[...]
    h = jnp.maximum(h, 0.0)
    rows = lax.broadcasted_iota(jnp.int32, (NP, 1), 0)
    mask = (rows < N).astype(jnp.float32)
    out_ref[...] = jnp.sum(h * mask, axis=0, keepdims=True) * (1.0 / N)


_tc2 = pl.pallas_call(
    _tc2_body,
    out_shape=jax.ShapeDtypeStruct((1, H), jnp.float32),
)


def kernel(node_feat, edge_index, emb_table, W1, b1, W2, b2):
    src = edge_index[0].astype(jnp.int32)
    dst = edge_index[1].astype(jnp.int32)
    pad_ix = N + (jnp.arange(EP - E, dtype=jnp.int32) % (NP - N))
    src_p = jnp.concatenate([src, pad_ix])
    dst_p = jnp.concatenate([dst, pad_ix])
    nf_p = jnp.pad(node_feat.astype(jnp.int32), (0, NP - N))

    edges_a = jnp.stack([src_p, dst_p]).reshape(2, 16, NCH_T, CH)
    deg = _deg_kernel(edges_a)
    norms = _norm(deg)
    onorm = norms[0]
    inorm = norms[1].reshape(NP, 1)

    src_w = src_p.reshape(32, EPW)
    dst_w = dst_p.reshape(32, EPW)
    zeros_s = jnp.zeros((SSTRIPE,), jnp.float32)
    s_parts = _hist_kernel(src_w, dst_w, nf_p, onorm, zeros_s)
    s_parts = s_parts.reshape(2, NP, T)

    h1s = _tc1(s_parts, emb_table, W1, b1.reshape(1, H), inorm,
               onorm.reshape(NP, 1))

    src_c = src_p.reshape(32, NCH_W, CH)
    dst_c = dst_p.reshape(32, NCH_W, CH)
    zeros_nd = jnp.zeros((STRIPE, H), jnp.float32)
    agg = _agg_kernel(src_c, dst_c, h1s, zeros_nd)

    return _tc2(agg, W2, b2.reshape(1, H), inorm)


# speedup vs baseline: 2.1585x; 1.2853x over previous
import functools

import jax
import jax.numpy as jnp
from jax import lax
from jax.experimental import pallas as pl
from jax.experimental.pallas import tpu as pltpu
from jax.experimental.pallas import tpu_sc as plsc

N = 10000
NP = 10240
T = 64
H = 128
E = 320000
CH = 128
NCH_W = 80
EPW = NCH_W * CH
EP = 32 * EPW
NCH_T = EP // 16 // CH
STRIPE = NP // 16
SFLAT = NP * T
SSTRIPE = SFLAT // 16

_mesh = plsc.VectorSubcoreMesh(core_axis_name="c", subcore_axis_name="s")
_sc_params = pltpu.CompilerParams(needs_layout_passes=False)


@functools.partial(
    pl.kernel,
    out_type=jax.ShapeDtypeStruct((2, NP), jnp.float32),
    mesh=_mesh,
    compiler_params=_sc_params,
    scratch_types=[
        pltpu.VMEM_SHARED((NP,), jnp.float32),
        pltpu.VMEM((NCH_T, CH), jnp.int32),
        pltpu.VMEM((CH,), jnp.float32),
        pltpu.VMEM((STRIPE,), jnp.float32),
    ],
)
def _deg_kernel(edges_hbm, out_hbm, acc, eidx, ones_v, zeros_v):
    cid = lax.axis_index("c")
    sid = lax.axis_index("s")
    for j in range(CH // 16):
        ones_v[pl.ds(16 * j, 16)] = jnp.full((16,), 1.0, jnp.float32)
    for j in range(STRIPE // 16):
        zeros_v[pl.ds(16 * j, 16)] = jnp.zeros((16,), jnp.float32)
    pltpu.sync_copy(zeros_v, acc.at[pl.ds(sid * STRIPE, STRIPE)])
    pltpu.sync_copy(edges_hbm.at[cid, sid], eidx)
    plsc.subcore_barrier()

    def body(c, carry):
        pltpu.sync_copy(ones_v, acc.at[eidx.at[c]], add=True)
        return carry

    lax.fori_loop(0, NCH_T, body, 0)
    plsc.subcore_barrier()
    pltpu.sync_copy(acc.at[pl.ds(sid * STRIPE, STRIPE)],
                    out_hbm.at[cid, pl.ds(sid * STRIPE, STRIPE)])


@functools.partial(
    pl.kernel,
    out_type=jax.ShapeDtypeStruct((2, SFLAT), jnp.float32),
    mesh=_mesh,
    compiler_params=_sc_params,
    scratch_types=[
        pltpu.VMEM_SHARED((SFLAT,), jnp.float32),
        pltpu.VMEM((NP,), jnp.int32),
        pltpu.VMEM((NP,), jnp.float32),
        pltpu.VMEM((EPW,), jnp.int32),
        pltpu.VMEM((EPW,), jnp.int32),
        pltpu.VMEM((NCH_W, CH), jnp.int32),
        pltpu.VMEM((EPW,), jnp.float32),
    ],
)
def _hist_kernel(src_hbm, dst_hbm, nf_hbm, onorm_hbm, z_hbm, out_hbm,
                 s_acc, t_v, w_v, src_v, dst_v, idx_v, val_v):
    cid = lax.axis_index("c")
    sid = lax.axis_index("s")
    wid = cid * 16 + sid
    pltpu.sync_copy(z_hbm, s_acc.at[pl.ds(sid * SSTRIPE, SSTRIPE)])
    pltpu.sync_copy(nf_hbm, t_v)
    pltpu.sync_copy(onorm_hbm, w_v)
    pltpu.sync_copy(src_hbm.at[wid], src_v)
    pltpu.sync_copy(dst_hbm.at[wid], dst_v)
    plsc.subcore_barrier()

    def body(c, carry):
        for j in range(CH // 16):
            s16 = src_v[pl.ds(c * CH + j * 16, 16)]
            d16 = dst_v[pl.ds(c * CH + j * 16, 16)]
            t16 = plsc.load_gather(t_v, [s16])
            w16 = plsc.load_gather(w_v, [s16])
            idx_v[c, pl.ds(j * 16, 16)] = d16 * T + t16
            val_v[pl.ds(c * CH + j * 16, 16)] = w16
        pltpu.sync_copy(val_v.at[pl.ds(c * CH, CH)],
                        s_acc.at[idx_v.at[c]], add=True)
        return carry

    lax.fori_loop(0, NCH_W, body, 0)
    plsc.subcore_barrier()
    pltpu.sync_copy(s_acc.at[pl.ds(sid * SSTRIPE, SSTRIPE)],
                    out_hbm.at[cid, pl.ds(sid * SSTRIPE, SSTRIPE)])


WCH = 40


@functools.partial(
    pl.kernel,
    out_type=jax.ShapeDtypeStruct((2, NP, H), jnp.float32),
    mesh=_mesh,
    compiler_params=_sc_params,
    scratch_types=[
        pltpu.VMEM_SHARED((NP, H), jnp.float32),
        pltpu.VMEM((WCH, CH), jnp.int32),
        pltpu.VMEM((WCH, CH), jnp.int32),
        pltpu.VMEM((CH, H), jnp.float32),
        pltpu.VMEM((CH, H), jnp.float32),
        pltpu.SemaphoreType.DMA,
        pltpu.SemaphoreType.DMA,
    ],
)
def _agg_kernel(src_hbm, dst_hbm, h1s_hbm, z_hbm, out_hbm,
                agc, src_v, dst_v, rows_a, rows_b, sem_a, sem_b):
    cid = lax.axis_index("c")
    sid = lax.axis_index("s")
    wid = cid * 16 + sid
    pltpu.sync_copy(z_hbm, agc.at[pl.ds(sid * STRIPE, STRIPE)])
    plsc.subcore_barrier()

    @pl.loop(0, 2)
    def _win(w):
        pltpu.sync_copy(src_hbm.at[wid, pl.ds(w * WCH, WCH)], src_v)
        pltpu.sync_copy(dst_hbm.at[wid, pl.ds(w * WCH, WCH)], dst_v)
        pltpu.async_copy(h1s_hbm.at[src_v.at[0]], rows_a, sem_a)

        @pl.loop(0, WCH - 2, step=2)
        def _pair(c):
            pltpu.async_copy(h1s_hbm.at[src_v.at[c + 1]], rows_b, sem_b)
            pltpu.make_async_copy(
                h1s_hbm.at[src_v.at[c]], rows_a, sem_a).wait()
            pltpu.sync_copy(rows_a, agc.at[dst_v.at[c]], add=True)
            pltpu.async_copy(h1s_hbm.at[src_v.at[c + 2]], rows_a, sem_a)
            pltpu.make_async_copy(
                h1s_hbm.at[src_v.at[c + 1]], rows_b, sem_b).wait()
            pltpu.sync_copy(rows_b, agc.at[dst_v.at[c + 1]], add=True)

        pltpu.async_copy(h1s_hbm.at[src_v.at[WCH - 1]], rows_b, sem_b)
        pltpu.make_async_copy(
            h1s_hbm.at[src_v.at[WCH - 2]], rows_a, sem_a).wait()
        pltpu.sync_copy(rows_a, agc.at[dst_v.at[WCH - 2]], add=True)
        pltpu.make_async_copy(
            h1s_hbm.at[src_v.at[WCH - 1]], rows_b, sem_b).wait()
        pltpu.sync_copy(rows_b, agc.at[dst_v.at[WCH - 1]], add=True)

    plsc.subcore_barrier()
    pltpu.sync_copy(agc.at[pl.ds(sid * STRIPE, STRIPE)],
                    out_hbm.at[cid, pl.ds(sid * STRIPE, STRIPE)])


def _norm_body(deg_ref, out_ref):
    out_ref[...] = lax.rsqrt(jnp.maximum(deg_ref[...], 1.0))


_norm = pl.pallas_call(
    _norm_body,
    out_shape=jax.ShapeDtypeStruct((2, NP), jnp.float32),
)


def _tc1_body(s_ref, e_ref, w1_ref, b1_ref, inorm_ref, onorm_ref, h1s_ref):
    s = s_ref[0] + s_ref[1]
    agg = jnp.dot(s, e_ref[...], preferred_element_type=jnp.float32)
    h = agg * inorm_ref[...]
    h = jnp.dot(h, w1_ref[...], preferred_element_type=jnp.float32) + b1_ref[...]
    h = jnp.maximum(h, 0.0)
    h1s_ref[...] = h * onorm_ref[...]


_tc1 = pl.pallas_call(
    _tc1_body,
    out_shape=jax.ShapeDtypeStruct((NP, H), jnp.float32),
)


def _tc2_body(a_ref, w2_ref, b2_ref, inorm_ref, out_ref):
    a = a_ref[0] + a_ref[1]
    h = a * inorm_ref[...]
    h = jnp.dot(h, w2_ref[...], preferred_element_type=jnp.float32) + b2_ref[...]
    h = jnp.maximum(h, 0.0)
    rows = lax.broadcasted_iota(jnp.int32, (NP, 1), 0)
    mask = (rows < N).astype(jnp.float32)
    out_ref[...] = jnp.sum(h * mask, axis=0, keepdims=True) * (1.0 / N)


_tc2 = pl.pallas_call(
    _tc2_body,
    out_shape=jax.ShapeDtypeStruct((1, H), jnp.float32),
)


def kernel(node_feat, edge_index, emb_table, W1, b1, W2, b2):
    src = edge_index[0].astype(jnp.int32)
    dst = edge_index[1].astype(jnp.int32)
    pad_ix = N + (jnp.arange(EP - E, dtype=jnp.int32) % (NP - N))
    src_p = jnp.concatenate([src, pad_ix])
    dst_p = jnp.concatenate([dst, pad_ix])
    nf_p = jnp.pad(node_feat.astype(jnp.int32), (0, NP - N))

    edges_a = jnp.stack([src_p, dst_p]).reshape(2, 16, NCH_T, CH)
    deg = _deg_kernel(edges_a)
    norms = _norm(deg)
    onorm = norms[0]
    inorm = norms[1].reshape(NP, 1)

    src_w = src_p.reshape(32, EPW)
    dst_w = dst_p.reshape(32, EPW)
    zeros_s = jnp.zeros((SSTRIPE,), jnp.float32)
    s_parts = _hist_kernel(src_w, dst_w, nf_p, onorm, zeros_s)
    s_parts = s_parts.reshape(2, NP, T)

    h1s = _tc1(s_parts, emb_table, W1, b1.reshape(1, H), inorm,
               onorm.reshape(NP, 1))

    src_c = src_p.reshape(32, 2 * WCH, CH)
    dst_c = dst_p.reshape(32, 2 * WCH, CH)
    zeros_nd = jnp.zeros((STRIPE, H), jnp.float32)
    agg = _agg_kernel(src_c, dst_c, h1s, zeros_nd)

    return _tc2(agg, W2, b2.reshape(1, H), inorm)
